# baseline (speedup 1.0000x reference)
import functools

import jax
import jax.numpy as jnp
from jax import lax
from jax.experimental import pallas as pl
from jax.experimental.pallas import tpu as pltpu
from jax.experimental.pallas import tpu_sc as plsc

_CH = 128


def _sc_gather(x, row, col):
  n, feat = x.shape
  e = row.shape[0]
  info = plsc.get_sparse_core_info()
  nc, ns = info.num_cores, info.num_subcores
  nw = nc * ns
  n_chunks = e // _CH
  iters = (n_chunks + nw - 1) // nw

  mesh = plsc.VectorSubcoreMesh(core_axis_name="c", subcore_axis_name="s")

  @functools.partial(
      pl.kernel,
      mesh=mesh,
      out_type=(jax.ShapeDtypeStruct((e, feat), jnp.float32),
                jax.ShapeDtypeStruct((e, feat), jnp.float32)),
      scratch_types=[
          pltpu.VMEM((_CH,), jnp.int32),
          pltpu.VMEM((_CH,), jnp.int32),
          pltpu.VMEM((_CH, feat), jnp.float32),
          pltpu.VMEM((_CH, feat), jnp.float32),
          pltpu.SemaphoreType.DMA,
          pltpu.SemaphoreType.DMA,
      ],
      compiler_params=pltpu.CompilerParams(use_tc_tiling_on_sc=False),
  )
  def k(x_hbm, row_hbm, col_hbm, src_out, dst_out,
        idx_r, idx_c, rows_r, rows_c, sem_r, sem_c):
    wid = lax.axis_index("s") * nc + lax.axis_index("c")

    def body(i, carry):
      chunk = wid + i * nw

      @pl.when(chunk < n_chunks)
      def _():
        base = chunk * _CH
        pltpu.sync_copy(row_hbm.at[pl.ds(base, _CH)], idx_r)
        pltpu.sync_copy(col_hbm.at[pl.ds(base, _CH)], idx_c)
        cp_r = pltpu.async_copy(x_hbm.at[idx_r], rows_r, sem_r)
        cp_c = pltpu.async_copy(x_hbm.at[idx_c], rows_c, sem_c)
        cp_r.wait()
        cp_c.wait()
        pltpu.sync_copy(rows_r, src_out.at[pl.ds(base, _CH)])
        pltpu.sync_copy(rows_c, dst_out.at[pl.ds(base, _CH)])

      return carry

    lax.fori_loop(0, iters, body, 0)

  return k(x, row, col)


def _tc_edge_mlp(src, dst, ea, mlp1_params):
  (w1, b1), (w2, b2), (w3, b3) = mlp1_params
  e = src.shape[0]
  h = w1.shape[1]
  blk = 4000
  grid = e // blk

  zpad = jnp.zeros((16, h), jnp.float32)
  a_w = zpad.at[0:3].set(-w1[0:3]).at[15].set(w1[8])
  b_w = zpad.at[0:3].set(w1[0:3]).at[15].set(w1[9])
  c_w = w1[4:8]
  wn = w1[3].reshape(1, h)
  b1p = b1.reshape(1, h)
  b2p = b2.reshape(1, h)
  w3p = jnp.concatenate([w3, jnp.zeros((h, 4), jnp.float32)], axis=1)
  b3p = jnp.concatenate(
      [b3, jnp.array([1.0, 0.0, 0.0, 0.0], jnp.float32)]).reshape(1, 8)
  m3 = (jnp.arange(16) < 3).astype(jnp.float32).reshape(16, 1)
  pad48 = jnp.concatenate(
      [jnp.eye(4, dtype=jnp.float32), jnp.zeros((4, 4), jnp.float32)], axis=1)

  def body(src_ref, dst_ref, ea_ref, a_ref, b_ref, c_ref, wn_ref, b1_ref,
           w2_ref, b2_ref, w3_ref, b3_ref, m3_ref, p48_ref, out_ref):
    s = src_ref[...]
    d = dst_ref[...]
    att = ea_ref[...]
    q = d - s
    ssq = jnp.dot(q * q, m3_ref[...], preferred_element_type=jnp.float32)
    nrm = jnp.sqrt(ssq + 1e-12)
    pre = (jnp.dot(s, a_ref[...], preferred_element_type=jnp.float32)
           + jnp.dot(d, b_ref[...], preferred_element_type=jnp.float32)
           + jnp.dot(att, c_ref[...], preferred_element_type=jnp.float32)
           + nrm * wn_ref[...] + b1_ref[...])
    hh = jnp.maximum(pre, 0.0)
    hh = jnp.maximum(
        jnp.dot(hh, w2_ref[...], preferred_element_type=jnp.float32)
        + b2_ref[...], 0.0)
    oo = (jnp.dot(hh, w3_ref[...], preferred_element_type=jnp.float32)
          + b3_ref[...])
    out_ref[...] = oo + jnp.dot(att, p48_ref[...],
                                preferred_element_type=jnp.float32)

  wspec = lambda shape: pl.BlockSpec(shape, lambda i: (0, 0))
  return pl.pallas_call(
      body,
      grid=(grid,),
      in_specs=[
          pl.BlockSpec((blk, 16), lambda i: (i, 0)),
          pl.BlockSpec((blk, 16), lambda i: (i, 0)),
          pl.BlockSpec((blk, 4), lambda i: (i, 0)),
          wspec((16, h)), wspec((16, h)), wspec((4, h)), wspec((1, h)),
          wspec((1, h)),
          wspec((h, h)), wspec((1, h)),
          wspec((h, 8)), wspec((1, 8)),
          wspec((16, 1)), wspec((4, 8)),
      ],
      out_specs=pl.BlockSpec((blk, 8), lambda i: (i, 0)),
      out_shape=jax.ShapeDtypeStruct((e, 8), jnp.float32),
  )(src, dst, ea, a_w, b_w, c_w, wn, b1p, w2, b2p, w3p, b3p, m3, pad48)


def _sc_scatter(e8, col, n_pad):
  e = e8.shape[0]
  info = plsc.get_sparse_core_info()
  nc, ns = info.num_cores, info.num_subcores
  nw = nc * ns
  n_chunks = e // _CH
  iters = (n_chunks + nw - 1) // nw
  rows_per_tile = n_pad // ns

  zeros8 = jnp.zeros((n_pad, 8), jnp.float32)
  mesh = plsc.VectorSubcoreMesh(core_axis_name="c", subcore_axis_name="s")

  @functools.partial(
      pl.kernel,
      mesh=mesh,
      out_type=jax.ShapeDtypeStruct((nc, n_pad, 8), jnp.float32),
      scratch_types=[
          pltpu.VMEM((_CH,), jnp.int32),
          pltpu.VMEM((_CH, 8), jnp.float32),
          pltpu.VMEM_SHARED((n_pad, 8), jnp.float32),
      ],
      compiler_params=pltpu.CompilerParams(use_tc_tiling_on_sc=False),
  )
  def k(e_hbm, col_hbm, z_hbm, out_hbm, idx_v, ev, acc):
    cid = lax.axis_index("c")
    sid = lax.axis_index("s")
    wid = sid * nc + cid
    r0 = sid * rows_per_tile

    pltpu.sync_copy(z_hbm.at[pl.ds(r0, rows_per_tile)],
                    acc.at[pl.ds(r0, rows_per_tile)])
    plsc.subcore_barrier()

    def body(i, carry):
      chunk = wid + i * nw

      @pl.when(chunk < n_chunks)
      def _():
        base = chunk * _CH
        pltpu.sync_copy(col_hbm.at[pl.ds(base, _CH)], idx_v)
        pltpu.sync_copy(e_hbm.at[pl.ds(base, _CH)], ev)
        pltpu.sync_copy(ev, acc.at[idx_v], add=True)

      return carry

    lax.fori_loop(0, iters, body, 0)
    plsc.subcore_barrier()

    pltpu.sync_copy(acc.at[pl.ds(r0, rows_per_tile)],
                    out_hbm.at[cid].at[pl.ds(r0, rows_per_tile)])

  return k(e8, col, zeros8)


def _tc_node_dec(x, p0, p1, mlp2_params, dec_params, mode):
  (w21, b21), (w22, b22), (w23, b23) = mlp2_params
  n = x.shape[0]
  h = w21.shape[1]
  t = dec_params[-1][0].shape[1]
  blk = 2000
  grid = n // blk

  wx = jnp.zeros((16, h), jnp.float32).at[14:16].set(w21[0:2])
  wa = jnp.zeros((8, h), jnp.float32).at[0:4].set(w21[2:6])
  sel4 = jnp.zeros((8, 1), jnp.float32).at[4, 0].set(1.0)
  dec_flat = []
  for (wd, bd) in dec_params:
    dec_flat.append(wd)
    dec_flat.append(bd.reshape(1, -1))
  mode_arr = jnp.reshape(jnp.asarray(mode, jnp.int32), (1, 1))

  def body(x_ref, p0_ref, p1_ref, wx_ref, wa_ref, sel_ref, b21_ref, w22_ref,
           b22_ref, w23_ref, b23_ref, d1_ref, db1_ref, d2_ref, db2_ref,
           d3_ref, db3_ref, d4_ref, db4_ref, mode_ref, out_ref):
    xx = x_ref[...]
    ps = p0_ref[...] + p1_ref[...]
    cnt = jnp.maximum(
        jnp.dot(ps, sel_ref[...], preferred_element_type=jnp.float32), 1.0)
    ps_scaled = ps / cnt
    hh = jnp.maximum(
        jnp.dot(xx, wx_ref[...], preferred_element_type=jnp.float32)
        + jnp.dot(ps_scaled, wa_ref[...], preferred_element_type=jnp.float32)
        + b21_ref[...], 0.0)
    hh = jnp.maximum(
        jnp.dot(hh, w22_ref[...], preferred_element_type=jnp.float32)
        + b22_ref[...], 0.0)
    delta = (jnp.dot(hh, w23_ref[...], preferred_element_type=jnp.float32)
             + b23_ref[...])
    lastcol = (lax.broadcasted_iota(jnp.int32, (1, 16), 1) == 15)
    x_res = xx + delta * lastcol.astype(jnp.float32)
    x_new = xx + jnp.maximum(x_res, 0.0)
    hh = jnp.maximum(
        jnp.dot(x_new, d1_ref[...], preferred_element_type=jnp.float32)
        + db1_ref[...], 0.0)
    hh = jnp.maximum(
        jnp.dot(hh, d2_ref[...], preferred_element_type=jnp.float32)
        + db2_ref[...], 0.0)
    hh = jnp.maximum(
        jnp.dot(hh, d3_ref[...], preferred_element_type=jnp.float32)
        + db3_ref[...], 0.0)
    oo = (jnp.dot(hh, d4_ref[...], preferred_element_type=jnp.float32)
          + db4_ref[...])
    mask = (mode_ref[0, 0] == 1).astype(jnp.float32)
    out_ref[...] = oo * mask

  wspec = lambda shape: pl.BlockSpec(shape, lambda i: (0, 0))
  return pl.pallas_call(
      body,
      grid=(grid,),
      in_specs=[
          pl.BlockSpec((blk, 16), lambda i: (i, 0)),
          pl.BlockSpec((blk, 8), lambda i: (i, 0)),
          pl.BlockSpec((blk, 8), lambda i: (i, 0)),
          wspec((16, h)), wspec((8, h)), wspec((8, 1)), wspec((1, h)),
          wspec((h, h)), wspec((1, h)),
          wspec((h, 1)), wspec((1, 1)),
          wspec((16, h)), wspec((1, h)),
          wspec((h, h)), wspec((1, h)),
          wspec((h, h)), wspec((1, h)),
          wspec((h, t)), wspec((1, t)),
          pl.BlockSpec(memory_space=pltpu.SMEM),
      ],
      out_specs=pl.BlockSpec((blk, t), lambda i: (i, 0)),
      out_shape=jax.ShapeDtypeStruct((n, t), jnp.float32),
  )(x, p0, p1, wx, wa, sel4, b21.reshape(1, h), w22, b22.reshape(1, h),
    w23, b23.reshape(1, 1), *dec_flat, mode_arr)


def kernel(x, edge_index, edge_attr, mlp1_params, mlp2_params, dec_params,
           mode):
  n = x.shape[0]
  n_pad = ((n + _CH - 1) // _CH) * _CH
  row = edge_index[0]
  col = edge_index[1]
  src_rows, dst_rows = _sc_gather(x, row, col)
  e8 = _tc_edge_mlp(src_rows, dst_rows, edge_attr, mlp1_params)
  parts = _sc_scatter(e8, col, n_pad)
  p0 = parts[0, :n]
  p1 = parts[1, :n]
  return _tc_node_dec(x, p0, p1, mlp2_params, dec_params, mode)


# speedup vs baseline: 1.5153x; 1.5153x over previous
import functools

import jax
import jax.numpy as jnp
from jax import lax
from jax.experimental import pallas as pl
from jax.experimental.pallas import tpu as pltpu
from jax.experimental.pallas import tpu_sc as plsc

_CH = 128


_G = 10


def _sc_gather(x, row, col):
  n, feat = x.shape
  e = row.shape[0]
  info = plsc.get_sparse_core_info()
  nc, ns = info.num_cores, info.num_subcores
  nw = nc * ns
  ss_edges = _G * _CH
  n_ss = e // ss_edges
  iters = (n_ss + nw - 1) // nw

  mesh = plsc.VectorSubcoreMesh(core_axis_name="c", subcore_axis_name="s")

  @functools.partial(
      pl.kernel,
      mesh=mesh,
      out_type=(jax.ShapeDtypeStruct((e, feat), jnp.float32),
                jax.ShapeDtypeStruct((e, feat), jnp.float32)),
      scratch_types=[
          pltpu.VMEM((_G, _CH), jnp.int32),
          pltpu.VMEM((_G, _CH), jnp.int32),
          pltpu.VMEM((ss_edges, feat), jnp.float32),
          pltpu.VMEM((ss_edges, feat), jnp.float32),
          pltpu.SemaphoreType.DMA,
          pltpu.SemaphoreType.DMA,
          pltpu.SemaphoreType.DMA,
      ],
      compiler_params=pltpu.CompilerParams(use_tc_tiling_on_sc=False),
  )
  def k(x_hbm, row_hbm, col_hbm, src_out, dst_out,
        idx_r, idx_c, rows_r, rows_c, sem_i, sem_r, sem_c):
    wid = lax.axis_index("s") * nc + lax.axis_index("c")

    def body(i, carry):
      ss = wid + i * nw

      @pl.when(ss < n_ss)
      def _():
        base = ss * ss_edges
        ld_r = pltpu.async_copy(
            row_hbm.at[pl.ds(ss * _G, _G)], idx_r, sem_i)
        ld_c = pltpu.async_copy(
            col_hbm.at[pl.ds(ss * _G, _G)], idx_c, sem_i)
        ld_r.wait()
        ld_c.wait()
        cps = []
        for g in range(_G):
          cps.append(pltpu.async_copy(
              x_hbm.at[idx_r.at[g]], rows_r.at[pl.ds(g * _CH, _CH)], sem_r))
          cps.append(pltpu.async_copy(
              x_hbm.at[idx_c.at[g]], rows_c.at[pl.ds(g * _CH, _CH)], sem_c))
        for cp in cps:
          cp.wait()
        pltpu.sync_copy(rows_r, src_out.at[pl.ds(base, ss_edges)])
        pltpu.sync_copy(rows_c, dst_out.at[pl.ds(base, ss_edges)])

      return carry

    lax.fori_loop(0, iters, body, 0)

  return k(x, row.reshape(e // _CH, _CH), col.reshape(e // _CH, _CH))


def _tc_edge_mlp(src, dst, ea, mlp1_params):
  (w1, b1), (w2, b2), (w3, b3) = mlp1_params
  e = src.shape[0]
  h = w1.shape[1]

  zpad = jnp.zeros((16, h), jnp.float32)
  a_w = zpad.at[0:3].set(-w1[0:3]).at[15].set(w1[8])
  b_w = zpad.at[0:3].set(w1[0:3]).at[15].set(w1[9])
  c_w = w1[4:8]
  wn = w1[3].reshape(1, h)
  w3p = jnp.concatenate([w3, jnp.zeros((h, 12), jnp.float32)], axis=1)
  b3p = jnp.concatenate(
      [b3, jnp.zeros((12,), jnp.float32)]).reshape(1, 16).at[0, 4].set(1.0)
  m3 = (jnp.arange(16) < 3).astype(jnp.float32).reshape(16, 1)
  pad4_16 = jnp.concatenate(
      [jnp.eye(4, dtype=jnp.float32), jnp.zeros((4, 12), jnp.float32)],
      axis=1)

  def bd(w, p=8):
    a, b = w.shape
    out = jnp.zeros((p * a, p * b), jnp.float32)
    for i in range(p):
      out = out.at[i * a:(i + 1) * a, i * b:(i + 1) * b].set(w)
    return out

  a8 = bd(a_w)
  b8w = bd(b_w)
  c8 = bd(c_w)
  wn8 = bd(wn)
  m38 = bd(m3)
  w28 = bd(w2)
  w38 = bd(w3p)
  p48 = bd(pad4_16)
  b18 = jnp.tile(b1.reshape(1, h), (1, 8))
  b28 = jnp.tile(b2.reshape(1, h), (1, 8))
  b38 = jnp.tile(b3p, (1, 8))

  e8r = e // 8
  blk = 1000
  grid = e8r // blk
  h8 = 8 * h

  def body(src_ref, dst_ref, ea_ref, a_ref, b_ref, c_ref, wn_ref, m3_ref,
           b1_ref, w2_ref, b2_ref, w3_ref, b3_ref, p48_ref, out_ref):
    s = src_ref[...]
    d = dst_ref[...]
    att = ea_ref[...]
    q = d - s
    ssq = jnp.dot(q * q, m3_ref[...], preferred_element_type=jnp.float32)
    nrm = jnp.sqrt(ssq + 1e-12)
    pre = (jnp.dot(s, a_ref[...], preferred_element_type=jnp.float32)
           + jnp.dot(d, b_ref[...], preferred_element_type=jnp.float32)
           + jnp.dot(att, c_ref[...], preferred_element_type=jnp.float32)
           + jnp.dot(nrm, wn_ref[...], preferred_element_type=jnp.float32)
           + b1_ref[...])
    hh = jnp.maximum(pre, 0.0)
    hh = jnp.maximum(
        jnp.dot(hh, w2_ref[...], preferred_element_type=jnp.float32)
        + b2_ref[...], 0.0)
    oo = (jnp.dot(hh, w3_ref[...], preferred_element_type=jnp.float32)
          + b3_ref[...])
    out_ref[...] = oo + jnp.dot(att, p48_ref[...],
                                preferred_element_type=jnp.float32)

  wspec = lambda shape: pl.BlockSpec(shape, lambda i: (0, 0))
  out8 = pl.pallas_call(
      body,
      grid=(grid,),
      in_specs=[
          pl.BlockSpec((blk, 128), lambda i: (i, 0)),
          pl.BlockSpec((blk, 128), lambda i: (i, 0)),
          pl.BlockSpec((blk, 32), lambda i: (i, 0)),
          wspec((128, h8)), wspec((128, h8)), wspec((32, h8)),
          wspec((8, h8)), wspec((128, 8)),
          wspec((1, h8)),
          wspec((h8, h8)), wspec((1, h8)),
          wspec((h8, 128)), wspec((1, 128)),
          wspec((32, 128)),
      ],
      out_specs=pl.BlockSpec((blk, 128), lambda i: (i, 0)),
      out_shape=jax.ShapeDtypeStruct((e8r, 128), jnp.float32),
  )(src.reshape(e8r, 128), dst.reshape(e8r, 128), ea.reshape(e8r, 32),
    a8, b8w, c8, wn8, m38, b18, w28, b28, w38, b38, p48)
  return out8.reshape(e, 16)


def _sc_scatter(e8, col, n_pad):
  e = e8.shape[0]
  info = plsc.get_sparse_core_info()
  nc, ns = info.num_cores, info.num_subcores
  nw = nc * ns
  ss_edges = _G * _CH
  n_ss = e // ss_edges
  iters = (n_ss + nw - 1) // nw
  rows_per_tile = n_pad // ns

  zeros16 = jnp.zeros((n_pad, 16), jnp.float32)
  mesh = plsc.VectorSubcoreMesh(core_axis_name="c", subcore_axis_name="s")

  @functools.partial(
      pl.kernel,
      mesh=mesh,
      out_type=jax.ShapeDtypeStruct((nc, n_pad, 16), jnp.float32),
      scratch_types=[
          pltpu.VMEM((_G, _CH), jnp.int32),
          pltpu.VMEM((ss_edges, 16), jnp.float32),
          pltpu.VMEM_SHARED((n_pad, 16), jnp.float32),
          pltpu.SemaphoreType.DMA,
          pltpu.SemaphoreType.DMA,
      ],
      compiler_params=pltpu.CompilerParams(use_tc_tiling_on_sc=False),
  )
  def k(e_hbm, col_hbm, z_hbm, out_hbm, idx_v, ev, acc, sem_i, sem_s):
    cid = lax.axis_index("c")
    sid = lax.axis_index("s")
    wid = sid * nc + cid
    r0 = sid * rows_per_tile

    pltpu.sync_copy(z_hbm.at[pl.ds(r0, rows_per_tile)],
                    acc.at[pl.ds(r0, rows_per_tile)])
    plsc.subcore_barrier()

    def body(i, carry):
      ss = wid + i * nw

      @pl.when(ss < n_ss)
      def _():
        base = ss * ss_edges
        ld_i = pltpu.async_copy(col_hbm.at[pl.ds(ss * _G, _G)], idx_v, sem_i)
        ld_e = pltpu.async_copy(e_hbm.at[pl.ds(base, ss_edges)], ev, sem_i)
        ld_i.wait()
        ld_e.wait()
        cps = []
        for g in range(_G):
          cps.append(pltpu.async_copy(
              ev.at[pl.ds(g * _CH, _CH)], acc.at[idx_v.at[g]], sem_s,
              add=True))
        for cp in cps:
          cp.wait()

      return carry

    lax.fori_loop(0, iters, body, 0)
    plsc.subcore_barrier()

    pltpu.sync_copy(acc.at[pl.ds(r0, rows_per_tile)],
                    out_hbm.at[cid].at[pl.ds(r0, rows_per_tile)])

  return k(e8, col.reshape(e // _CH, _CH), zeros16)


def _tc_node_dec(x, p0, p1, mlp2_params, dec_params, mode):
  (w21, b21), (w22, b22), (w23, b23) = mlp2_params
  n = x.shape[0]
  h = w21.shape[1]
  t = dec_params[-1][0].shape[1]
  blk = 2000
  grid = n // blk

  wx = jnp.zeros((16, h), jnp.float32).at[14:16].set(w21[0:2])
  wa = jnp.zeros((16, h), jnp.float32).at[0:4].set(w21[2:6])
  sel4 = jnp.zeros((16, 1), jnp.float32).at[4, 0].set(1.0)
  dec_flat = []
  for (wd, bd) in dec_params:
    dec_flat.append(wd)
    dec_flat.append(bd.reshape(1, -1))
  mode_arr = jnp.reshape(jnp.asarray(mode, jnp.int32), (1, 1))

  def body(x_ref, p0_ref, p1_ref, wx_ref, wa_ref, sel_ref, b21_ref, w22_ref,
           b22_ref, w23_ref, b23_ref, d1_ref, db1_ref, d2_ref, db2_ref,
           d3_ref, db3_ref, d4_ref, db4_ref, mode_ref, out_ref):
    xx = x_ref[...]
    ps = p0_ref[...] + p1_ref[...]
    cnt = jnp.maximum(
        jnp.dot(ps, sel_ref[...], preferred_element_type=jnp.float32), 1.0)
    ps_scaled = ps / cnt
    hh = jnp.maximum(
        jnp.dot(xx, wx_ref[...], preferred_element_type=jnp.float32)
        + jnp.dot(ps_scaled, wa_ref[...], preferred_element_type=jnp.float32)
        + b21_ref[...], 0.0)
    hh = jnp.maximum(
        jnp.dot(hh, w22_ref[...], preferred_element_type=jnp.float32)
        + b22_ref[...], 0.0)
    delta = (jnp.dot(hh, w23_ref[...], preferred_element_type=jnp.float32)
             + b23_ref[...])
    lastcol = (lax.broadcasted_iota(jnp.int32, (1, 16), 1) == 15)
    x_res = xx + delta * lastcol.astype(jnp.float32)
    x_new = xx + jnp.maximum(x_res, 0.0)
    hh = jnp.maximum(
        jnp.dot(x_new, d1_ref[...], preferred_element_type=jnp.float32)
        + db1_ref[...], 0.0)
    hh = jnp.maximum(
        jnp.dot(hh, d2_ref[...], preferred_element_type=jnp.float32)
        + db2_ref[...], 0.0)
    hh = jnp.maximum(
        jnp.dot(hh, d3_ref[...], preferred_element_type=jnp.float32)
        + db3_ref[...], 0.0)
    oo = (jnp.dot(hh, d4_ref[...], preferred_element_type=jnp.float32)
          + db4_ref[...])
    mask = (mode_ref[0, 0] == 1).astype(jnp.float32)
    out_ref[...] = oo * mask

  wspec = lambda shape: pl.BlockSpec(shape, lambda i: (0, 0))
  return pl.pallas_call(
      body,
      grid=(grid,),
      in_specs=[
          pl.BlockSpec((blk, 16), lambda i: (i, 0)),
          pl.BlockSpec((blk, 16), lambda i: (i, 0)),
          pl.BlockSpec((blk, 16), lambda i: (i, 0)),
          wspec((16, h)), wspec((16, h)), wspec((16, 1)), wspec((1, h)),
          wspec((h, h)), wspec((1, h)),
          wspec((h, 1)), wspec((1, 1)),
          wspec((16, h)), wspec((1, h)),
          wspec((h, h)), wspec((1, h)),
          wspec((h, h)), wspec((1, h)),
          wspec((h, t)), wspec((1, t)),
          pl.BlockSpec(memory_space=pltpu.SMEM),
      ],
      out_specs=pl.BlockSpec((blk, t), lambda i: (i, 0)),
      out_shape=jax.ShapeDtypeStruct((n, t), jnp.float32),
  )(x, p0, p1, wx, wa, sel4, b21.reshape(1, h), w22, b22.reshape(1, h),
    w23, b23.reshape(1, 1), *dec_flat, mode_arr)


def kernel(x, edge_index, edge_attr, mlp1_params, mlp2_params, dec_params,
           mode):
  n = x.shape[0]
  n_pad = ((n + _CH - 1) // _CH) * _CH
  row = edge_index[0]
  col = edge_index[1]
  src_rows, dst_rows = _sc_gather(x, row, col)
  e16 = _tc_edge_mlp(src_rows, dst_rows, edge_attr, mlp1_params)
  parts = _sc_scatter(e16, col, n_pad)
  p0 = parts[0, :n]
  p1 = parts[1, :n]
  return _tc_node_dec(x, p0, p1, mlp2_params, dec_params, mode)
